# baseline (speedup 1.0000x reference)
import jax
import jax.numpy as jnp
from jax.experimental import pallas as pl
from jax.experimental.pallas import tpu as pltpu

_ROWS = 65536
_D = 32
_CBLK = 65536


def _matmul_t_block(xt_ref, w_ref, b_ref, o_ref):
    yt = jax.lax.dot_general(
        w_ref[...], xt_ref[...],
        dimension_numbers=(((0,), (0,)), ((), ())),
        preferred_element_type=jnp.float32,
    )
    o_ref[...] = yt + jax.lax.broadcast_in_dim(b_ref[...], (_D, _CBLK), (0,))


def kernel(x, W, b):
    xt = x.T
    yt = pl.pallas_call(
        _matmul_t_block,
        grid=(_ROWS // _CBLK,),
        in_specs=[
            pl.BlockSpec((_D, _CBLK), lambda i: (0, i)),
            pl.BlockSpec((_D, _D), lambda i: (0, 0)),
            pl.BlockSpec((_D,), lambda i: (0,)),
        ],
        out_specs=pl.BlockSpec((_D, _CBLK), lambda i: (0, i)),
        out_shape=jax.ShapeDtypeStruct((_D, _ROWS), jnp.float32),
        compiler_params=pltpu.CompilerParams(
            dimension_semantics=("arbitrary",),
        ),
    )(xt, W, b)
    return yt.T


# speedup vs baseline: 1.0923x; 1.0923x over previous
import jax
import jax.numpy as jnp
from jax.experimental import pallas as pl
from jax.experimental.pallas import tpu as pltpu

_ROWS = 65536
_D = 32
_HALF = _ROWS // 2


def _body(xt_hbm, w_ref, b_ref, yt_hbm, va, vb, oa, ob, sa, sb, soa, sob):
    cin_a = pltpu.make_async_copy(xt_hbm.at[:, pl.ds(0, _HALF)], va, sa)
    cin_b = pltpu.make_async_copy(xt_hbm.at[:, pl.ds(_HALF, _HALF)], vb, sb)
    cin_a.start()
    cin_b.start()

    bias = jax.lax.broadcast_in_dim(b_ref[...], (_D, _HALF), (0,))

    cin_a.wait()
    oa[...] = jax.lax.dot_general(
        w_ref[...], va[...],
        dimension_numbers=(((0,), (0,)), ((), ())),
        preferred_element_type=jnp.float32,
    ) + bias
    cout_a = pltpu.make_async_copy(oa, yt_hbm.at[:, pl.ds(0, _HALF)], soa)
    cout_a.start()

    cin_b.wait()
    ob[...] = jax.lax.dot_general(
        w_ref[...], vb[...],
        dimension_numbers=(((0,), (0,)), ((), ())),
        preferred_element_type=jnp.float32,
    ) + bias
    cout_b = pltpu.make_async_copy(ob, yt_hbm.at[:, pl.ds(_HALF, _HALF)], sob)
    cout_b.start()

    cout_a.wait()
    cout_b.wait()


def kernel(x, W, b):
    xt = x.T
    yt = pl.pallas_call(
        _body,
        in_specs=[
            pl.BlockSpec(memory_space=pltpu.MemorySpace.HBM),
            pl.BlockSpec(memory_space=pltpu.MemorySpace.VMEM),
            pl.BlockSpec(memory_space=pltpu.MemorySpace.VMEM),
        ],
        out_specs=pl.BlockSpec(memory_space=pltpu.MemorySpace.HBM),
        out_shape=jax.ShapeDtypeStruct((_D, _ROWS), jnp.float32),
        scratch_shapes=[
            pltpu.VMEM((_D, _HALF), jnp.float32),
            pltpu.VMEM((_D, _HALF), jnp.float32),
            pltpu.VMEM((_D, _HALF), jnp.float32),
            pltpu.VMEM((_D, _HALF), jnp.float32),
            pltpu.SemaphoreType.DMA,
            pltpu.SemaphoreType.DMA,
            pltpu.SemaphoreType.DMA,
            pltpu.SemaphoreType.DMA,
        ],
    )(xt, W, b)
    return yt.T


# speedup vs baseline: 1.1776x; 1.0782x over previous
import jax
import jax.numpy as jnp
from jax.experimental import pallas as pl
from jax.experimental.pallas import tpu as pltpu

_ROWS = 65536
_D = 32
_NCHUNK = 4
_CW = _ROWS // _NCHUNK


def _body(xt_hbm, w_ref, b_ref, yt_hbm, *rest):
    ins = rest[0:_NCHUNK]
    outs = rest[_NCHUNK:2 * _NCHUNK]
    sin = rest[2 * _NCHUNK:3 * _NCHUNK]
    sout = rest[3 * _NCHUNK:4 * _NCHUNK]

    cin = [
        pltpu.make_async_copy(xt_hbm.at[:, pl.ds(i * _CW, _CW)], ins[i], sin[i])
        for i in range(_NCHUNK)
    ]
    for c in cin:
        c.start()

    bias = jax.lax.broadcast_in_dim(b_ref[...], (_D, _CW), (0,))

    cout = []
    for i in range(_NCHUNK):
        cin[i].wait()
        outs[i][...] = jax.lax.dot_general(
            w_ref[...], ins[i][...],
            dimension_numbers=(((0,), (0,)), ((), ())),
            preferred_element_type=jnp.float32,
        ) + bias
        c = pltpu.make_async_copy(outs[i], yt_hbm.at[:, pl.ds(i * _CW, _CW)], sout[i])
        c.start()
        cout.append(c)

    for c in cout:
        c.wait()


def kernel(x, W, b):
    xt = x.T
    yt = pl.pallas_call(
        _body,
        in_specs=[
            pl.BlockSpec(memory_space=pltpu.MemorySpace.HBM),
            pl.BlockSpec(memory_space=pltpu.MemorySpace.VMEM),
            pl.BlockSpec(memory_space=pltpu.MemorySpace.VMEM),
        ],
        out_specs=pl.BlockSpec(memory_space=pltpu.MemorySpace.HBM),
        out_shape=jax.ShapeDtypeStruct((_D, _ROWS), jnp.float32),
        scratch_shapes=(
            [pltpu.VMEM((_D, _CW), jnp.float32) for _ in range(2 * _NCHUNK)]
            + [pltpu.SemaphoreType.DMA for _ in range(2 * _NCHUNK)]
        ),
    )(xt, W, b)
    return yt.T


# speedup vs baseline: 1.1834x; 1.0049x over previous
import jax
import jax.numpy as jnp
from jax.experimental import pallas as pl
from jax.experimental.pallas import tpu as pltpu

_ROWS = 65536
_D = 32
_NCHUNK = 8
_CW = _ROWS // _NCHUNK


def _body(xt_hbm, w_ref, b_ref, yt_hbm, *rest):
    ins = rest[0:_NCHUNK]
    outs = rest[_NCHUNK:2 * _NCHUNK]
    sin = rest[2 * _NCHUNK:3 * _NCHUNK]
    sout = rest[3 * _NCHUNK:4 * _NCHUNK]

    cin = [
        pltpu.make_async_copy(xt_hbm.at[:, pl.ds(i * _CW, _CW)], ins[i], sin[i])
        for i in range(_NCHUNK)
    ]
    for c in cin:
        c.start()

    bias = jax.lax.broadcast_in_dim(b_ref[...], (_D, _CW), (0,))

    cout = []
    for i in range(_NCHUNK):
        cin[i].wait()
        outs[i][...] = jax.lax.dot_general(
            w_ref[...], ins[i][...],
            dimension_numbers=(((0,), (0,)), ((), ())),
            preferred_element_type=jnp.float32,
        ) + bias
        c = pltpu.make_async_copy(outs[i], yt_hbm.at[:, pl.ds(i * _CW, _CW)], sout[i])
        c.start()
        cout.append(c)

    for c in cout:
        c.wait()


def kernel(x, W, b):
    xt = x.T
    yt = pl.pallas_call(
        _body,
        in_specs=[
            pl.BlockSpec(memory_space=pltpu.MemorySpace.HBM),
            pl.BlockSpec(memory_space=pltpu.MemorySpace.VMEM),
            pl.BlockSpec(memory_space=pltpu.MemorySpace.VMEM),
        ],
        out_specs=pl.BlockSpec(memory_space=pltpu.MemorySpace.HBM),
        out_shape=jax.ShapeDtypeStruct((_D, _ROWS), jnp.float32),
        scratch_shapes=(
            [pltpu.VMEM((_D, _CW), jnp.float32) for _ in range(2 * _NCHUNK)]
            + [pltpu.SemaphoreType.DMA for _ in range(2 * _NCHUNK)]
        ),
    )(xt, W, b)
    return yt.T


# speedup vs baseline: 1.2266x; 1.0365x over previous
import jax
import jax.numpy as jnp
from jax.experimental import pallas as pl
from jax.experimental.pallas import tpu as pltpu

_ROWS = 65536
_D = 32
_CBLK = 32768


def _matmul_t_block(xt_ref, w_ref, b_ref, o_ref):
    yt = jax.lax.dot_general(
        w_ref[...], xt_ref[...],
        dimension_numbers=(((0,), (0,)), ((), ())),
        preferred_element_type=jnp.float32,
    )
    o_ref[...] = yt + jax.lax.broadcast_in_dim(b_ref[...], (_D, _CBLK), (0,))


def kernel(x, W, b):
    xt = x.T
    yt = pl.pallas_call(
        _matmul_t_block,
        grid=(_ROWS // _CBLK,),
        in_specs=[
            pl.BlockSpec((_D, _CBLK), lambda i: (0, i)),
            pl.BlockSpec((_D, _D), lambda i: (0, 0)),
            pl.BlockSpec((_D,), lambda i: (0,)),
        ],
        out_specs=pl.BlockSpec((_D, _CBLK), lambda i: (0, i)),
        out_shape=jax.ShapeDtypeStruct((_D, _ROWS), jnp.float32),
        compiler_params=pltpu.CompilerParams(
            dimension_semantics=("arbitrary",),
        ),
    )(xt, W, b)
    return yt.T
